# baseline (speedup 1.0000x reference)
import functools

import jax
import jax.numpy as jnp
from jax.experimental import pallas as pl

DIM = 64
M_DIM = 16
HID = 2 * (2 * DIM + 2)


def _silu(x):
    return 0.5 * x * (1.0 + jnp.tanh(0.5 * x))


def _hsilu(u):
    t = jnp.tanh(u)
    return u + u * t


def _egnn_block_kernel(
    fti_ref, fta_ref,
    cmi_ref, cmT_ref, cvi_ref, cvT_ref, aug_ref,
    w1a_ref, w1b_ref, wdm_ref, wds_ref, b1_ref,
    w2_ref, b2_ref,
    hw1_ref, hb1_ref, hw2_ref, hb2_ref,
    nw1_ref, nb1_ref, nw2_ref, nb2_ref,
    node_out_ref, cm_out_ref, cv_out_ref,
    *, bi, n,
):
    fti = jnp.transpose(fti_ref[0])
    fta = jnp.transpose(fta_ref[0])

    at = jnp.dot(w1a_ref[...], fti, preferred_element_type=jnp.float32)
    at = (at + b1_ref[...]).astype(jnp.bfloat16)
    bt = jnp.dot(w1b_ref[...], fta,
                 preferred_element_type=jnp.float32).astype(jnp.bfloat16)

    cm_i = cmi_ref[0]
    cv_i = cvi_ref[0]
    cmT = jnp.transpose(cmT_ref[0])
    cvT = jnp.transpose(cvT_ref[0])
    dsum = jnp.zeros((bi, n), jnp.float32)
    vtr = jnp.zeros((bi, n), jnp.float32)
    q = jnp.zeros((bi, n), jnp.float32)
    for c in range(3):
        rel = cm_i[:, c:c + 1] - cmT[c:c + 1, :]
        rv = cv_i[:, c:c + 1] + cvT[c:c + 1, :]
        rel2 = rel * rel
        dsum = dsum + rel2
        vtr = vtr + rv
        q = q + rel2 * rv
    dm = (dsum + vtr).astype(jnp.bfloat16)
    ds = (2.0 * vtr + 4.0 * q).astype(jnp.bfloat16)

    aug = aug_ref[0]
    wdm = wdm_ref[...]
    wds = wds_ref[...]
    w2 = w2_ref[...]

    hs = []
    mts = []
    for i in range(bi):
        pre = (at[:, i:i + 1] + bt
               + dm[i:i + 1, :] * wdm
               + ds[i:i + 1, :] * wds)
        hs.append(_hsilu(pre))
        if i >= 1:
            mts.append(jnp.dot(w2, hs[i - 1],
                               preferred_element_type=jnp.float32))
    mts.append(jnp.dot(w2, hs[-1], preferred_element_type=jnp.float32))
    mt_l = jnp.concatenate(mts, axis=1).astype(jnp.bfloat16)
    mt_l = _hsilu(mt_l + b2_ref[...])

    hh = jnp.dot(hw1_ref[...], mt_l,
                 preferred_element_type=jnp.float32).astype(jnp.bfloat16)
    hh = _hsilu(hh + hb1_ref[...])
    wo = jnp.dot(hw2_ref[...], hh,
                 preferred_element_type=jnp.float32) + hb2_ref[...]
    wo_b = wo.astype(jnp.bfloat16)
    wo2 = (wo * wo).astype(jnp.bfloat16)

    cmat = jnp.concatenate(
        [wo_b[:, n * i:n * (i + 1)] for i in range(bi)]
        + [wo2[:, n * i:n * (i + 1)] for i in range(bi)]
        + [mt_l[:, n * i:n * (i + 1)] for i in range(bi)],
        axis=0)
    s = jnp.dot(cmat, aug, preferred_element_type=jnp.float32)
    w8 = jnp.concatenate(
        [s[2 * i:2 * i + 1, :] for i in range(bi)], axis=0)
    v8 = jnp.concatenate(
        [s[2 * bi + 2 * i + 1:2 * bi + 2 * i + 2, :] for i in range(bi)],
        axis=0)
    moff = 4 * bi
    m_t = jnp.concatenate(
        [s[moff + M_DIM * i:moff + M_DIM * (i + 1), 3:4] for i in range(bi)],
        axis=1)

    cm_out = cm_i + w8[:, 3:4] * cm_i - w8[:, 0:3]
    cv_out = cv_i + v8[:, 7:8] * cv_i + v8[:, 4:7]

    nint = jnp.concatenate([fti, m_t], axis=0)
    nh = _hsilu(jnp.dot(nw1_ref[...], nint,
                        preferred_element_type=jnp.float32) + nb1_ref[...])
    nout = jnp.dot(nw2_ref[...], nh,
                   preferred_element_type=jnp.float32) + nb2_ref[...] + fti

    node_out_ref[0] = nout.T
    cm_out_ref[0] = cm_out
    cv_out_ref[0] = cv_out


@jax.jit
def kernel(feats, coors_mean, coors_var, params):
    b, n, d = feats.shape
    bi = 16

    w1 = params['edge_w1']
    w1a = 0.5 * w1[:, :DIM]
    w1b = 0.5 * w1[:, DIM:2 * DIM]
    wdm = (0.5 * w1[:, 2 * DIM:2 * DIM + 1]).astype(jnp.bfloat16)
    wds = (0.5 * w1[:, 2 * DIM + 1:2 * DIM + 2]).astype(jnp.bfloat16)
    b1 = 0.5 * params['edge_b1'].reshape(HID, 1)
    w2 = (0.5 * params['edge_w2']).astype(jnp.bfloat16)
    b2 = (0.5 * params['edge_b2'].reshape(M_DIM, 1)).astype(jnp.bfloat16)
    hw1 = (0.5 * jnp.concatenate(
        [params['cm_w1'], params['cv_w1']], axis=0)).astype(jnp.bfloat16)
    hb1 = (0.5 * jnp.concatenate(
        [params['cm_b1'], params['cv_b1']]).reshape(8 * M_DIM, 1)
           ).astype(jnp.bfloat16)
    z64 = jnp.zeros((1, 4 * M_DIM), jnp.float32)
    hw2 = jnp.concatenate([
        jnp.concatenate([params['cm_w2'], z64], axis=1),
        jnp.concatenate([z64, params['cv_w2']], axis=1),
    ], axis=0).astype(jnp.bfloat16)
    hb2 = jnp.concatenate(
        [params['cm_b2'], params['cv_b2']]).reshape(2, 1)
    nw1 = 0.5 * params['node_w1']
    nb1 = 0.5 * params['node_b1'].reshape(2 * DIM, 1)
    nw2 = params['node_w2']
    nb2 = params['node_b2'].reshape(DIM, 1)

    ones = jnp.ones((b, n, 1), jnp.float32)
    aug = jnp.concatenate(
        [coors_mean, ones, coors_var, ones], axis=2).astype(jnp.bfloat16)

    grid = (b, n // bi)

    def im_block(ib, ii):
        return (ib, ii, 0)

    def im_batch(ib, ii):
        return (ib, 0, 0)

    def im_const(ib, ii):
        return (0, 0)

    full = lambda shape: pl.BlockSpec(shape, im_const)

    out_shapes = (
        jax.ShapeDtypeStruct((b, n, d), jnp.float32),
        jax.ShapeDtypeStruct((b, n, 3), jnp.float32),
        jax.ShapeDtypeStruct((b, n, 3), jnp.float32),
    )

    node_out, cm_out, cv_out = pl.pallas_call(
        functools.partial(_egnn_block_kernel, bi=bi, n=n),
        grid=grid,
        in_specs=[
            pl.BlockSpec((1, bi, d), im_block),
            pl.BlockSpec((1, n, d), im_batch),
            pl.BlockSpec((1, bi, 3), im_block),
            pl.BlockSpec((1, n, 3), im_batch),
            pl.BlockSpec((1, bi, 3), im_block),
            pl.BlockSpec((1, n, 3), im_batch),
            pl.BlockSpec((1, n, 8), im_batch),
            full((HID, DIM)), full((HID, DIM)),
            full((HID, 1)), full((HID, 1)), full((HID, 1)),
            full((M_DIM, HID)), full((M_DIM, 1)),
            full((8 * M_DIM, M_DIM)), full((8 * M_DIM, 1)),
            full((2, 8 * M_DIM)), full((2, 1)),
            full((2 * DIM, DIM + M_DIM)), full((2 * DIM, 1)),
            full((DIM, 2 * DIM)), full((DIM, 1)),
        ],
        out_specs=(
            pl.BlockSpec((1, bi, d), im_block),
            pl.BlockSpec((1, bi, 3), im_block),
            pl.BlockSpec((1, bi, 3), im_block),
        ),
        out_shape=out_shapes,
    )(
        feats, feats,
        coors_mean, coors_mean, coors_var, coors_var, aug,
        w1a, w1b, wdm, wds, b1,
        w2, b2,
        hw1, hb1, hw2, hb2,
        nw1, nb1, nw2, nb2,
    )
    return node_out, cm_out, cv_out


# speedup vs baseline: 1.0812x; 1.0812x over previous
import functools

import jax
import jax.numpy as jnp
from jax.experimental import pallas as pl

DIM = 64
M_DIM = 16
HID = 2 * (2 * DIM + 2)


def _silu(x):
    return 0.5 * x * (1.0 + jnp.tanh(0.5 * x))


def _hsilu(u):
    t = jnp.tanh(u)
    return u + u * t


def _egnn_block_kernel(
    fti_ref, fta_ref,
    cmi_ref, cmT_ref, cvi_ref, cvT_ref, aug_ref,
    w1_ref, b1_ref, w2_ref, b2_ref,
    cmw1_ref, cmb1_ref, cmw2_ref, cmb2_ref,
    cvw1_ref, cvb1_ref, cvw2_ref, cvb2_ref,
    nw1_ref, nb1_ref, nw2_ref, nb2_ref,
    node_out_ref, cm_out_ref, cv_out_ref,
    *, bi, n,
):
    fti = fti_ref[0, 0]
    fta = fta_ref[0]

    w1 = w1_ref[...]
    b1 = b1_ref[...]
    wdm = (0.5 * w1[:, 2 * DIM:2 * DIM + 1]).astype(jnp.bfloat16)
    wds = (0.5 * w1[:, 2 * DIM + 1:2 * DIM + 2]).astype(jnp.bfloat16)
    w2 = (0.5 * w2_ref[...]).astype(jnp.bfloat16)
    b2 = (0.5 * b2_ref[...]).astype(jnp.bfloat16)
    hw1 = (0.5 * jnp.concatenate([cmw1_ref[...], cvw1_ref[...]], axis=0)
           ).astype(jnp.bfloat16)
    hb1 = (0.5 * jnp.concatenate([cmb1_ref[...], cvb1_ref[...]], axis=0)
           ).astype(jnp.bfloat16)
    z64 = jnp.zeros((1, 4 * M_DIM), jnp.float32)
    hw2 = jnp.concatenate([
        jnp.concatenate([cmw2_ref[...], z64], axis=1),
        jnp.concatenate([z64, cvw2_ref[...]], axis=1),
    ], axis=0).astype(jnp.bfloat16)
    hb2 = jnp.concatenate([cmb2_ref[...], cvb2_ref[...]], axis=0)

    at = jnp.dot(w1[:, :DIM], 0.5 * fti,
                 preferred_element_type=jnp.float32)
    at = (at + 0.5 * b1).astype(jnp.bfloat16)
    bt = jnp.dot(w1[:, DIM:2 * DIM], 0.5 * fta,
                 preferred_element_type=jnp.float32).astype(jnp.bfloat16)

    cm_i = cmi_ref[0]
    cv_i = cvi_ref[0]
    cmT = cmT_ref[0]
    cvT = cvT_ref[0]
    dsum = jnp.zeros((bi, n), jnp.float32)
    vtr = jnp.zeros((bi, n), jnp.float32)
    q = jnp.zeros((bi, n), jnp.float32)
    for c in range(3):
        rel = cm_i[:, c:c + 1] - cmT[c:c + 1, :]
        rv = cv_i[:, c:c + 1] + cvT[c:c + 1, :]
        rel2 = rel * rel
        dsum = dsum + rel2
        vtr = vtr + rv
        q = q + rel2 * rv
    dm = (dsum + vtr).astype(jnp.bfloat16)
    ds = (2.0 * vtr + 4.0 * q).astype(jnp.bfloat16)

    aug = aug_ref[0]

    hs = []
    mts = []
    for i in range(bi):
        pre = (at[:, i:i + 1] + bt
               + dm[i:i + 1, :] * wdm
               + ds[i:i + 1, :] * wds)
        hs.append(_hsilu(pre))
        if i >= 1:
            mts.append(jnp.dot(w2, hs[i - 1],
                               preferred_element_type=jnp.float32))
    mts.append(jnp.dot(w2, hs[-1], preferred_element_type=jnp.float32))
    mt_l = jnp.concatenate(mts, axis=1).astype(jnp.bfloat16)
    mt_l = _hsilu(mt_l + b2)

    hh = jnp.dot(hw1, mt_l,
                 preferred_element_type=jnp.float32).astype(jnp.bfloat16)
    hh = _hsilu(hh + hb1)
    wo = jnp.dot(hw2, hh,
                 preferred_element_type=jnp.float32) + hb2
    wo_b = wo.astype(jnp.bfloat16)
    wo2 = (wo * wo).astype(jnp.bfloat16)

    cmat = jnp.concatenate(
        [wo_b[:, n * i:n * (i + 1)] for i in range(bi)]
        + [wo2[:, n * i:n * (i + 1)] for i in range(bi)]
        + [mt_l[:, n * i:n * (i + 1)] for i in range(bi)],
        axis=0)
    s = jnp.dot(cmat, aug, preferred_element_type=jnp.float32)
    w8 = jnp.concatenate(
        [s[2 * i:2 * i + 1, :] for i in range(bi)], axis=0)
    v8 = jnp.concatenate(
        [s[2 * bi + 2 * i + 1:2 * bi + 2 * i + 2, :] for i in range(bi)],
        axis=0)
    moff = 4 * bi
    m_t = jnp.concatenate(
        [s[moff + M_DIM * i:moff + M_DIM * (i + 1), 3:4] for i in range(bi)],
        axis=1)

    cm_out = cm_i + w8[:, 3:4] * cm_i - w8[:, 0:3]
    cv_out = cv_i + v8[:, 7:8] * cv_i + v8[:, 4:7]

    nint = jnp.concatenate([fti, m_t], axis=0)
    nh = _hsilu(jnp.dot(nw1_ref[...], 0.5 * nint,
                        preferred_element_type=jnp.float32)
                + 0.5 * nb1_ref[...])
    nout = jnp.dot(nw2_ref[...], nh,
                   preferred_element_type=jnp.float32) + nb2_ref[...] + fti

    node_out_ref[0] = nout.T
    cm_out_ref[0] = cm_out
    cv_out_ref[0] = cv_out


@jax.jit
def kernel(feats, coors_mean, coors_var, params):
    b, n, d = feats.shape
    bi = 16

    w1 = params['edge_w1']
    b1 = params['edge_b1'].reshape(HID, 1)
    w2 = params['edge_w2']
    b2 = params['edge_b2'].reshape(M_DIM, 1)
    cmw1 = params['cm_w1']
    cmb1 = params['cm_b1'].reshape(4 * M_DIM, 1)
    cmw2 = params['cm_w2']
    cmb2 = params['cm_b2'].reshape(1, 1)
    cvw1 = params['cv_w1']
    cvb1 = params['cv_b1'].reshape(4 * M_DIM, 1)
    cvw2 = params['cv_w2']
    cvb2 = params['cv_b2'].reshape(1, 1)
    nw1 = params['node_w1']
    nb1 = params['node_b1'].reshape(2 * DIM, 1)
    nw2 = params['node_w2']
    nb2 = params['node_b2'].reshape(DIM, 1)

    featsT = jnp.transpose(feats, (0, 2, 1))
    featsT_blk = jnp.transpose(
        featsT.reshape(b, d, n // bi, bi), (0, 2, 1, 3))
    cmT = jnp.transpose(coors_mean, (0, 2, 1))
    cvT = jnp.transpose(coors_var, (0, 2, 1))
    ones = jnp.ones((b, n, 1), jnp.float32)
    aug = jnp.concatenate(
        [coors_mean, ones, coors_var, ones], axis=2).astype(jnp.bfloat16)

    grid = (b, n // bi)

    def im_block(ib, ii):
        return (ib, ii, 0)

    def im_icol(ib, ii):
        return (ib, ii, 0, 0)

    def im_batch(ib, ii):
        return (ib, 0, 0)

    def im_const(ib, ii):
        return (0, 0)

    full = lambda shape: pl.BlockSpec(shape, im_const)

    out_shapes = (
        jax.ShapeDtypeStruct((b, n, d), jnp.float32),
        jax.ShapeDtypeStruct((b, n, 3), jnp.float32),
        jax.ShapeDtypeStruct((b, n, 3), jnp.float32),
    )

    node_out, cm_out, cv_out = pl.pallas_call(
        functools.partial(_egnn_block_kernel, bi=bi, n=n),
        grid=grid,
        in_specs=[
            pl.BlockSpec((1, 1, d, bi), im_icol),
            pl.BlockSpec((1, d, n), im_batch),
            pl.BlockSpec((1, bi, 3), im_block),
            pl.BlockSpec((1, 3, n), im_batch),
            pl.BlockSpec((1, bi, 3), im_block),
            pl.BlockSpec((1, 3, n), im_batch),
            pl.BlockSpec((1, n, 8), im_batch),
            full((HID, 2 * DIM + 2)), full((HID, 1)),
            full((M_DIM, HID)), full((M_DIM, 1)),
            full((4 * M_DIM, M_DIM)), full((4 * M_DIM, 1)),
            full((1, 4 * M_DIM)), full((1, 1)),
            full((4 * M_DIM, M_DIM)), full((4 * M_DIM, 1)),
            full((1, 4 * M_DIM)), full((1, 1)),
            full((2 * DIM, DIM + M_DIM)), full((2 * DIM, 1)),
            full((DIM, 2 * DIM)), full((DIM, 1)),
        ],
        out_specs=(
            pl.BlockSpec((1, bi, d), im_block),
            pl.BlockSpec((1, bi, 3), im_block),
            pl.BlockSpec((1, bi, 3), im_block),
        ),
        out_shape=out_shapes,
    )(
        featsT_blk, featsT,
        coors_mean, cmT, coors_var, cvT, aug,
        w1, b1, w2, b2,
        cmw1, cmb1, cmw2, cmb2,
        cvw1, cvb1, cvw2, cvb2,
        nw1, nb1, nw2, nb2,
    )
    return node_out, cm_out, cv_out


# speedup vs baseline: 1.0816x; 1.0004x over previous
import functools

import jax
import jax.numpy as jnp
from jax.experimental import pallas as pl

DIM = 64
M_DIM = 16
HID = 2 * (2 * DIM + 2)


def _hsilu(u):
    t = jnp.tanh(u)
    return u + u * t


def _egnn_block_kernel(
    fti_ref, fta_ref,
    cmi_ref, cmT_ref, cvi_ref, cvT_ref, aug_ref,
    w1_ref, b1_ref, w2_ref, b2_ref,
    cmw1_ref, cmb1_ref, cmw2_ref, cmb2_ref,
    cvw1_ref, cvb1_ref, cvw2_ref, cvb2_ref,
    nw1_ref, nb1_ref, nw2_ref, nb2_ref,
    node_out_ref, cm_out_ref, cv_out_ref,
    *, bi, n,
):
    fti = fti_ref[0, 0]
    fta = fta_ref[0]

    w1 = w1_ref[...]
    b1 = b1_ref[...]
    wdm = (0.5 * w1[:, 2 * DIM:2 * DIM + 1]).astype(jnp.bfloat16)
    wds = (0.5 * w1[:, 2 * DIM + 1:2 * DIM + 2]).astype(jnp.bfloat16)
    w2 = (0.5 * w2_ref[...]).astype(jnp.bfloat16)
    b2 = (0.5 * b2_ref[...]).astype(jnp.bfloat16)
    hw1 = (0.5 * jnp.concatenate([cmw1_ref[...], cvw1_ref[...]], axis=0)
           ).astype(jnp.bfloat16)
    hb1 = (0.5 * jnp.concatenate([cmb1_ref[...], cvb1_ref[...]], axis=0)
           ).astype(jnp.bfloat16)
    z64 = jnp.zeros((1, 4 * M_DIM), jnp.float32)
    hw2 = jnp.concatenate([
        jnp.concatenate([cmw2_ref[...], z64], axis=1),
        jnp.concatenate([z64, cvw2_ref[...]], axis=1),
    ], axis=0).astype(jnp.bfloat16)
    hb2 = jnp.concatenate([cmb2_ref[...], cvb2_ref[...]], axis=0)

    at = jnp.dot(w1[:, :DIM], 0.5 * fti,
                 preferred_element_type=jnp.float32)
    at = (at + 0.5 * b1).astype(jnp.bfloat16)
    bt = jnp.dot(w1[:, DIM:2 * DIM], 0.5 * fta,
                 preferred_element_type=jnp.float32).astype(jnp.bfloat16)

    cm_i = cmi_ref[0]
    cv_i = cvi_ref[0]
    cmT = cmT_ref[0]
    cvT = cvT_ref[0]
    dsum = jnp.zeros((bi, n), jnp.float32)
    vtr = jnp.zeros((bi, n), jnp.float32)
    q = jnp.zeros((bi, n), jnp.float32)
    for c in range(3):
        rel = cm_i[:, c:c + 1] - cmT[c:c + 1, :]
        rv = cv_i[:, c:c + 1] + cvT[c:c + 1, :]
        rel2 = rel * rel
        dsum = dsum + rel2
        vtr = vtr + rv
        q = q + rel2 * rv
    dm = (dsum + vtr).astype(jnp.bfloat16)
    ds = (2.0 * vtr + 4.0 * q).astype(jnp.bfloat16)

    aug = aug_ref[0]

    hs = []
    mts = []
    for i in range(bi):
        pre = (at[:, i:i + 1] + bt
               + dm[i:i + 1, :] * wdm
               + ds[i:i + 1, :] * wds)
        hs.append(_hsilu(pre))
        if i >= 1:
            mts.append(jnp.dot(w2, hs[i - 1],
                               preferred_element_type=jnp.float32))
    mts.append(jnp.dot(w2, hs[-1], preferred_element_type=jnp.float32))
    mt_l = jnp.concatenate(mts, axis=1).astype(jnp.bfloat16)
    mt_l = _hsilu(mt_l + b2)

    hh = jnp.dot(hw1, mt_l,
                 preferred_element_type=jnp.float32).astype(jnp.bfloat16)
    hh = _hsilu(hh + hb1)
    wo = jnp.dot(hw2, hh,
                 preferred_element_type=jnp.float32) + hb2
    wo_b = wo.astype(jnp.bfloat16)
    wo2 = (wo * wo).astype(jnp.bfloat16)

    cmat = jnp.concatenate(
        [wo_b[:, n * i:n * (i + 1)] for i in range(bi)]
        + [wo2[:, n * i:n * (i + 1)] for i in range(bi)]
        + [mt_l[:, n * i:n * (i + 1)] for i in range(bi)],
        axis=0)
    s = jnp.dot(cmat, aug, preferred_element_type=jnp.float32)
    w8 = jnp.concatenate(
        [s[2 * i:2 * i + 1, :] for i in range(bi)], axis=0)
    v8 = jnp.concatenate(
        [s[2 * bi + 2 * i + 1:2 * bi + 2 * i + 2, :] for i in range(bi)],
        axis=0)
    moff = 4 * bi
    m_t = jnp.concatenate(
        [s[moff + M_DIM * i:moff + M_DIM * (i + 1), 3:4] for i in range(bi)],
        axis=1)

    cm_out = cm_i + w8[:, 3:4] * cm_i - w8[:, 0:3]
    cv_out = cv_i + v8[:, 7:8] * cv_i + v8[:, 4:7]

    nint = jnp.concatenate([fti, m_t], axis=0)
    nh = _hsilu(jnp.dot(nw1_ref[...], 0.5 * nint,
                        preferred_element_type=jnp.float32)
                + 0.5 * nb1_ref[...])
    nout = jnp.dot(nw2_ref[...], nh,
                   preferred_element_type=jnp.float32) + nb2_ref[...] + fti

    node_out_ref[0] = nout.T
    cm_out_ref[0] = cm_out
    cv_out_ref[0] = cv_out


@jax.jit
def kernel(feats, coors_mean, coors_var, params):
    b, n, d = feats.shape
    bi = 16

    w1 = params['edge_w1']
    b1 = params['edge_b1'].reshape(HID, 1)
    w2 = params['edge_w2']
    b2 = params['edge_b2'].reshape(M_DIM, 1)
    cmw1 = params['cm_w1']
    cmb1 = params['cm_b1'].reshape(4 * M_DIM, 1)
    cmw2 = params['cm_w2']
    cmb2 = params['cm_b2'].reshape(1, 1)
    cvw1 = params['cv_w1']
    cvb1 = params['cv_b1'].reshape(4 * M_DIM, 1)
    cvw2 = params['cv_w2']
    cvb2 = params['cv_b2'].reshape(1, 1)
    nw1 = params['node_w1']
    nb1 = params['node_b1'].reshape(2 * DIM, 1)
    nw2 = params['node_w2']
    nb2 = params['node_b2'].reshape(DIM, 1)

    featsT = jnp.transpose(feats, (0, 2, 1))
    featsT_blk = jnp.transpose(
        featsT.reshape(b, d, n // bi, bi), (0, 2, 1, 3))
    cmT = jnp.transpose(coors_mean, (0, 2, 1))
    cvT = jnp.transpose(coors_var, (0, 2, 1))
    ones = jnp.ones((b, n, 1), jnp.float32)
    aug = jnp.concatenate(
        [coors_mean, ones, coors_var, ones], axis=2).astype(jnp.bfloat16)

    grid = (b, n // bi)

    def im_block(ib, ii):
        return (ib, ii, 0)

    def im_icol(ib, ii):
        return (ib, ii, 0, 0)

    def im_batch(ib, ii):
        return (ib, 0, 0)

    def im_const(ib, ii):
        return (0, 0)

    full = lambda shape: pl.BlockSpec(shape, im_const)

    out_shapes = (
        jax.ShapeDtypeStruct((b, n, d), jnp.float32),
        jax.ShapeDtypeStruct((b, n, 3), jnp.float32),
        jax.ShapeDtypeStruct((b, n, 3), jnp.float32),
    )

    node_out, cm_out, cv_out = pl.pallas_call(
        functools.partial(_egnn_block_kernel, bi=bi, n=n),
        grid=grid,
        in_specs=[
            pl.BlockSpec((1, 1, d, bi), im_icol),
            pl.BlockSpec((1, d, n), im_batch),
            pl.BlockSpec((1, bi, 3), im_block),
            pl.BlockSpec((1, 3, n), im_batch),
            pl.BlockSpec((1, bi, 3), im_block),
            pl.BlockSpec((1, 3, n), im_batch),
            pl.BlockSpec((1, n, 8), im_batch),
            full((HID, 2 * DIM + 2)), full((HID, 1)),
            full((M_DIM, HID)), full((M_DIM, 1)),
            full((4 * M_DIM, M_DIM)), full((4 * M_DIM, 1)),
            full((1, 4 * M_DIM)), full((1, 1)),
            full((4 * M_DIM, M_DIM)), full((4 * M_DIM, 1)),
            full((1, 4 * M_DIM)), full((1, 1)),
            full((2 * DIM, DIM + M_DIM)), full((2 * DIM, 1)),
            full((DIM, 2 * DIM)), full((DIM, 1)),
        ],
        out_specs=(
            pl.BlockSpec((1, bi, d), im_block),
            pl.BlockSpec((1, bi, 3), im_block),
            pl.BlockSpec((1, bi, 3), im_block),
        ),
        out_shape=out_shapes,
    )(
        featsT_blk, featsT,
        coors_mean, cmT, coors_var, cvT, aug,
        w1, b1, w2, b2,
        cmw1, cmb1, cmw2, cmb2,
        cvw1, cvb1, cvw2, cvb2,
        nw1, nb1, nw2, nb2,
    )
    return node_out, cm_out, cv_out
